# Initial kernel scaffold; baseline (speedup 1.0000x reference)
import jax
import jax.numpy as jnp
from jax.experimental import pallas as pl


def kernel(x, edge_index, edge_attr, batch, coords, anatomical_class, Wl1, Wr1, b1, g1, bt1, a1, Wl2, Wr2, b2, g2, bt2, a2, B_table, posW, posb, p, lin1W, lin1b, lin2W, lin2b):
    raise NotImplementedError("write your pallas kernel here")


# speedup vs baseline: 1.8636x; 1.8636x over previous
import jax
import jax.numpy as jnp
from jax.experimental import pallas as pl

_H = 64
_NG = 8
_NA = 6
_BLK = 2000
_NEG_INF = float("-inf")


def _elu(v):
    return jnp.where(v > 0, v, jnp.exp(jnp.minimum(v, 0.0)) - 1.0)


def _onehot_g(batch2):
    return (batch2 == jax.lax.broadcasted_iota(jnp.int32, (1, _NG), 1)).astype(jnp.float32)


def _segT(w, v):
    return jax.lax.dot_general(w, v, (((0,), (0,)), ((), ())),
                               preferred_element_type=jnp.float32,
                               precision=jax.lax.Precision.HIGHEST)


def _dot(a, b):
    return jnp.dot(a, b, preferred_element_type=jnp.float32,
                   precision=jax.lax.Precision.HIGHEST)


def _class_select_matmul(h, mats_ref, anat2):
    acc = None
    for c in range(_NA):
        m = (anat2 == c).astype(jnp.float32)
        term = m * _dot(h, mats_ref[:, c * _H:(c + 1) * _H])
        acc = term if acc is None else acc + term
    return acc


def _row_spec(w):
    return pl.BlockSpec((_BLK, w), lambda i: (i, 0))


def _const_spec(shape):
    return pl.BlockSpec(shape, lambda i: tuple(0 for _ in shape))


def _k0_body(x_ref, cp_ref, wl_ref, wr_ref, pw_ref, pb_ref,
             y1_ref, z1_ref, e_ref):
    x = x_ref[...]
    y1_ref[...] = _dot(x, wl_ref[...])
    z1_ref[...] = _dot(x, wr_ref[...])
    e_ref[...] = _dot(cp_ref[...], pw_ref[...]) + pb_ref[...]


def _k1a_body(s1_ref, c1_ref, z1_ref, b1_ref, batch_ref,
              h_ref, sh_ref, sh2_ref):
    h = s1_ref[...] / jnp.maximum(c1_ref[...], 1.0) + z1_ref[...] + b1_ref[...]
    h_ref[...] = h
    oh = _onehot_g(batch_ref[...])

    @pl.when(pl.program_id(0) == 0)
    def _():
        sh_ref[...] = jnp.zeros_like(sh_ref)
        sh2_ref[...] = jnp.zeros_like(sh2_ref)

    sh_ref[...] += _segT(oh, h)
    sh2_ref[...] += _segT(oh, h * h)


def _k1b_body(h_ref, ae1_ref, batch_ref, anat_ref, mu_ref, inv_ref,
              g1_ref, bt1_ref, a1_ref, btc_ref, bc2_ref, pvec_ref,
              h2a_ref, sc_ref):
    oh = _onehot_g(batch_ref[...])
    out = h_ref[...] - a1_ref[...] * _dot(oh, mu_ref[...])
    h = g1_ref[...] * out * _dot(oh, inv_ref[...]) + bt1_ref[...]
    h = _elu(h)
    anat2 = anat_ref[...]
    d1 = _class_select_matmul(ae1_ref[...], btc_ref, anat2)
    hk = h + d1
    h_out = _class_select_matmul(hk, bc2_ref, anat2) \
        + jnp.sum(hk * d1, axis=1, keepdims=True)
    p = pvec_ref[...]
    pn = p / (jnp.sqrt(jnp.sum(p * p)) + 1e-12)
    sc = jnp.tanh(jnp.sum(h_out * pn, axis=1, keepdims=True))
    sc_ref[...] = sc
    h2a_ref[...] = h_out * sc


def _k2_body(ae2_ref, h2a_ref, anat_ref, btc_ref, bc2_ref, wl2_ref,
             q_ref, ql_ref):
    anat2 = anat_ref[...]
    d2 = _class_select_matmul(ae2_ref[...], btc_ref, anat2)
    hh = h2a_ref[...] + d2
    q = _class_select_matmul(hh, bc2_ref, anat2) \
        + jnp.sum(hh * d2, axis=1, keepdims=True)
    q_ref[...] = q
    ql_ref[...] = _dot(q, wl2_ref[...])


def _k3a_body(s2_ref, c2_ref, q_ref, wr2_ref, b2_ref, keep_ref, batch_ref,
              h2_ref, swh_ref, swh2_ref, cw_ref):
    h2 = s2_ref[...] / jnp.maximum(c2_ref[...], 1.0) \
        + _dot(q_ref[...], wr2_ref[...]) + b2_ref[...]
    h2_ref[...] = h2
    keep = keep_ref[...]
    oh = _onehot_g(batch_ref[...])
    ohw = oh * keep

    @pl.when(pl.program_id(0) == 0)
    def _():
        swh_ref[...] = jnp.zeros_like(swh_ref)
        swh2_ref[...] = jnp.zeros_like(swh2_ref)
        cw_ref[...] = jnp.zeros_like(cw_ref)

    swh_ref[...] += _segT(ohw, h2)
    swh2_ref[...] += _segT(ohw, h2 * h2)
    cw_ref[...] += jnp.sum(ohw, axis=0).reshape(_NG, 1)


def _k3b_body(h2_ref, keep_ref, batch_ref, mu_ref, inv_ref,
              g2_ref, bt2_ref, a2_ref,
              sp_ref, mx_ref):
    oh = _onehot_g(batch_ref[...])
    out = h2_ref[...] - a2_ref[...] * _dot(oh, mu_ref[...])
    he = g2_ref[...] * out * _dot(oh, inv_ref[...]) + bt2_ref[...]
    he = _elu(he)
    keep = keep_ref[...]
    keep_b = keep > 0.0

    @pl.when(pl.program_id(0) == 0)
    def _():
        sp_ref[...] = jnp.zeros_like(sp_ref)
        mx_ref[...] = jnp.full_like(mx_ref, _NEG_INF)

    sp_ref[...] += _segT(oh * keep, he)
    batch2 = batch_ref[...]
    maxs = []
    for g in range(_NG):
        m = (batch2 == g) & keep_b
        maxs.append(jnp.max(jnp.where(m, he, _NEG_INF), axis=0, keepdims=True))
    mx_ref[...] = jnp.maximum(mx_ref[...], jnp.concatenate(maxs, axis=0))


def _k3c_body(sp_ref, cw_ref, mx_ref, l1w_ref, l1b_ref, l2w_ref, l2b_ref,
              out_ref):
    cnt = jnp.maximum(cw_ref[...], 1.0)
    feat = jnp.concatenate([sp_ref[...] / cnt, mx_ref[...], sp_ref[...]], axis=1)
    o1 = jnp.maximum(_dot(feat, l1w_ref[...]) + l1b_ref[...], 0.0)
    o2 = _dot(o1, l2w_ref[...]) + l2b_ref[...]
    m = jnp.max(o2, axis=1, keepdims=True)
    s = o2 - m
    out_ref[...] = s - jnp.log(jnp.sum(jnp.exp(s), axis=1, keepdims=True))


@jax.jit
def kernel(x, edge_index, edge_attr, batch, coords, anatomical_class,
           Wl1, Wr1, b1, g1, bt1, a1, Wl2, Wr2, b2, g2, bt2, a2,
           B_table, posW, posb, p, lin1W, lin1b, lin2W, lin2b):
    n = x.shape[0]
    grid = (n // _BLK,)
    row, col = edge_index[0], edge_index[1]
    w_edge = edge_attr[:, 0]
    batch2 = batch.astype(jnp.int32).reshape(n, 1)
    anat2 = anatomical_class.astype(jnp.int32).reshape(n, 1)
    coords_p = jnp.zeros((n, 8), jnp.float32).at[:, :3].set(coords)
    posW_p = jnp.zeros((8, _H), jnp.float32).at[:3, :].set(posW)
    b3 = B_table.reshape(_NA, _H, _H)
    btc = b3.transpose(2, 0, 1).reshape(_H, _NA * _H)
    bc2 = b3.transpose(1, 0, 2).reshape(_H, _NA * _H)
    r1 = lambda v: v.reshape(1, -1)

    nh = jax.ShapeDtypeStruct((n, _H), jnp.float32)
    n1 = jax.ShapeDtypeStruct((n, 1), jnp.float32)
    g64 = jax.ShapeDtypeStruct((_NG, _H), jnp.float32)
    g1s = jax.ShapeDtypeStruct((_NG, 1), jnp.float32)
    cs = _const_spec

    y1, z1, e = pl.pallas_call(
        _k0_body, grid=grid,
        in_specs=[_row_spec(128), _row_spec(8), cs((128, _H)), cs((128, _H)),
                  cs((8, _H)), cs((1, _H))],
        out_specs=[_row_spec(_H)] * 3,
        out_shape=[nh] * 3,
    )(x, coords_p, Wl1, Wr1, posW_p, r1(posb))

    ones_e = jnp.ones((row.shape[0],), jnp.float32)
    s1 = jax.ops.segment_sum(y1[row], col, num_segments=n)
    c1 = jax.ops.segment_sum(ones_e, col, num_segments=n).reshape(n, 1)
    ae1 = jax.ops.segment_sum(w_edge[:, None] * e[col], row, num_segments=n)

    h, sh, sh2 = pl.pallas_call(
        _k1a_body, grid=grid,
        in_specs=[_row_spec(_H), _row_spec(1), _row_spec(_H), cs((1, _H)),
                  _row_spec(1)],
        out_specs=[_row_spec(_H), cs((_NG, _H)), cs((_NG, _H))],
        out_shape=[nh, g64, g64],
    )(s1, c1, z1, r1(b1), batch2)

    counts = jax.ops.segment_sum(jnp.ones((n,), jnp.int32), batch, num_segments=_NG)
    cnt_f = jnp.maximum(counts.astype(jnp.float32), 1.0).reshape(_NG, 1)
    mu = sh / cnt_f
    var = sh2 / cnt_f - (2.0 * a1 - a1 * a1) * mu * mu
    inv = 1.0 / jnp.sqrt(var + 1e-5)

    h2a, sc = pl.pallas_call(
        _k1b_body, grid=grid,
        in_specs=[_row_spec(_H), _row_spec(_H), _row_spec(1), _row_spec(1),
                  cs((_NG, _H)), cs((_NG, _H)), cs((1, _H)), cs((1, _H)),
                  cs((1, _H)), cs((_H, _NA * _H)), cs((_H, _NA * _H)),
                  cs((1, _H))],
        out_specs=[_row_spec(_H), _row_spec(1)],
        out_shape=[nh, n1],
    )(h, ae1, batch2, anat2, mu, inv, r1(g1), r1(bt1), r1(a1), btc, bc2, r1(p))

    scores = sc[:, 0]
    n_idx = jnp.arange(n, dtype=batch.dtype)
    kg = (7 * counts + 9) // 10
    starts = jnp.cumsum(counts) - counts
    b_s, _, order = jax.lax.sort((batch, -scores, n_idx), num_keys=2)
    rank = jnp.arange(n, dtype=jnp.int32) - starts[b_s]
    keep_sorted = rank < kg[b_s]
    keep = jnp.zeros((n,), jnp.bool_).at[order].set(keep_sorted)
    keep_f = keep.astype(jnp.float32)

    w2 = keep_f[row] * keep_f[col]
    ae2 = jax.ops.segment_sum(w2[:, None] * e[col], row, num_segments=n)

    q, ql = pl.pallas_call(
        _k2_body, grid=grid,
        in_specs=[_row_spec(_H), _row_spec(_H), _row_spec(1),
                  cs((_H, _NA * _H)), cs((_H, _NA * _H)), cs((_H, _H))],
        out_specs=[_row_spec(_H)] * 2,
        out_shape=[nh] * 2,
    )(ae2, h2a, anat2, btc, bc2, Wl2)

    s2 = jax.ops.segment_sum(w2[:, None] * ql[row], col, num_segments=n)
    c2 = jax.ops.segment_sum(w2, col, num_segments=n).reshape(n, 1)

    keep2 = keep_f.reshape(n, 1)
    h2, swh, swh2, cw = pl.pallas_call(
        _k3a_body, grid=grid,
        in_specs=[_row_spec(_H), _row_spec(1), _row_spec(_H), cs((_H, _H)),
                  cs((1, _H)), _row_spec(1), _row_spec(1)],
        out_specs=[_row_spec(_H), cs((_NG, _H)), cs((_NG, _H)), cs((_NG, 1))],
        out_shape=[nh, g64, g64, g1s],
    )(s2, c2, q, Wr2, r1(b2), keep2, batch2)

    cw_f = jnp.maximum(cw, 1.0)
    mu2 = swh / cw_f
    var2 = swh2 / cw_f - (2.0 * a2 - a2 * a2) * mu2 * mu2
    inv2 = 1.0 / jnp.sqrt(var2 + 1e-5)

    sp, mx = pl.pallas_call(
        _k3b_body, grid=grid,
        in_specs=[_row_spec(_H), _row_spec(1), _row_spec(1),
                  cs((_NG, _H)), cs((_NG, _H)), cs((1, _H)), cs((1, _H)),
                  cs((1, _H))],
        out_specs=[cs((_NG, _H)), cs((_NG, _H))],
        out_shape=[g64, g64],
    )(h2, keep2, batch2, mu2, inv2, r1(g2), r1(bt2), r1(a2))

    nc = lin2b.shape[0]
    out = pl.pallas_call(
        _k3c_body,
        out_shape=jax.ShapeDtypeStruct((_NG, nc), jnp.float32),
    )(sp, cw, mx, lin1W, r1(lin1b), lin2W, r1(lin2b))
    return out
